# Initial kernel scaffold; baseline (speedup 1.0000x reference)
import jax
import jax.numpy as jnp
from jax.experimental import pallas as pl


def kernel(src, pos, src_shape, src_mask, src_start_index, src_valid_ratios, ref_windows, W1, b1, W2, b2, ln_g, ln_b, bbox_w1, bbox_b1, bbox_w2, bbox_b2, bbox_w3, bbox_b3, cls_w, cls_b, enc_w, enc_b, enc_ln_g, enc_ln_b):
    raise NotImplementedError("write your pallas kernel here")


# speedup vs baseline: 2.0381x; 2.0381x over previous
import functools
import math

import jax
import jax.numpy as jnp
from jax import lax
from jax.experimental import pallas as pl
from jax.experimental.pallas import tpu as pltpu
from jax.experimental.pallas import tpu_sc as plsc

B, L, D, FFN, NSC, NQ, DEC = 4, 16384, 256, 1024, 4, 900, 256
NPAD = 1024
TA = 2048
HI = lax.Precision.HIGHEST
NEG = -65504.0


def _enc_body(src_ref, pos_ref, msk_ref, w1_ref, b1_ref, w2_ref, b2_ref,
              g_ref, bb_ref, cw_ref, cb_ref, out_ref, lg_ref):
    x = src_ref[...]
    q = x + pos_ref[...]
    h = jnp.maximum(jnp.dot(q, w1_ref[...], preferred_element_type=jnp.float32) + b1_ref[...], 0.0)
    h = jnp.dot(h, w2_ref[...], preferred_element_type=jnp.float32) + b2_ref[...]
    y = x + h
    yt = y.T
    mu = (jnp.sum(yt, axis=0, keepdims=True) / float(D)).T
    var = (jnp.sum((yt - mu.T) ** 2, axis=0, keepdims=True) / float(D)).T
    yn = (y - mu) / jnp.sqrt(var + 1e-5) * g_ref[...] + bb_ref[...]
    m = msk_ref[...]
    yn = jnp.where(m > 0.5, 0.0, yn)
    out_ref[...] = yn
    lg = jnp.dot(yn, cw_ref[...],  preferred_element_type=jnp.float32) + cb_ref[...]
    lg_ref[...] = jnp.where(m > 0.5, NEG, lg)


def _topk_body(lg_ref, idx_ref, gidx_ref, gidx2_ref):
    b = pl.program_id(0)
    i32 = jnp.int32
    f32 = jnp.float32
    v = lg_ref[0]
    kraw = lax.bitcast_convert_type(v, i32)
    key = jnp.where(kraw < 0, kraw ^ i32(0x7FFFFFFF), kraw)
    MINI = i32(-2147483648)

    def bit_step(j, tu):
        cand = tu | (i32(1) << (31 - j))
        cnt = jnp.sum((key >= (cand ^ MINI)).astype(i32))
        return jnp.where(cnt >= NQ, cand, tu)

    tu = lax.fori_loop(0, 32, bit_step, i32(0))
    kth = tu ^ MINI
    n1 = jnp.sum((key > kth).astype(i32))
    tneed = NQ - n1

    rr = lax.broadcasted_iota(i32, (256, 256), 0)
    cc = lax.broadcasted_iota(i32, (256, 256), 1)
    e = rr * 256 + cc
    tie = key == kth

    def e_step(_, lohi):
        lo, hi = lohi
        mid = (lo + hi) // 2
        cnt = jnp.sum((tie & (e <= mid)).astype(i32))
        ok = cnt >= tneed
        return (jnp.where(ok, lo, mid), jnp.where(ok, mid, hi))

    _, ecut = lax.fori_loop(0, 20, e_step, (i32(-2), i32(L * NSC - 1)))
    sel = (key > kth) | (tie & (e <= ecut))
    selF = sel.astype(f32)

    ustrict = (lax.broadcasted_iota(i32, (256, 256), 0)
               < lax.broadcasted_iota(i32, (256, 256), 1)).astype(f32)
    prow = jnp.dot(selF, ustrict, precision=HI,
                   preferred_element_type=f32)
    rowsum = jnp.sum(selF, axis=1, keepdims=True)
    lstrict = (lax.broadcasted_iota(i32, (256, 256), 0)
               > lax.broadcasted_iota(i32, (256, 256), 1)).astype(f32)
    rowoff = jnp.dot(lstrict, rowsum, precision=HI,
                     preferred_element_type=f32)

    tF = lax.broadcasted_iota(i32, (NPAD, 1), 0).astype(f32)
    r_of = jnp.sum((rowoff.T <= tF).astype(f32), axis=1, keepdims=True) - 1.0
    c256 = lax.broadcasted_iota(i32, (NPAD, 256), 1).astype(f32)
    oneh_r = (c256 == r_of).astype(f32)
    rowoff_t = jnp.dot(oneh_r, rowoff, precision=HI,
                       preferred_element_type=f32)
    k_t = tF - rowoff_t
    prow_t = jnp.dot(oneh_r, prow, precision=HI, preferred_element_type=f32)
    sel_t = jnp.dot(oneh_r, selF, precision=HI, preferred_element_type=f32)
    val_t = jnp.dot(oneh_r, v, precision=HI, preferred_element_type=f32)
    match = ((prow_t == k_t) & (sel_t > 0.5)).astype(f32)
    cv = jnp.sum(match * val_t, axis=1, keepdims=True)
    ce = jnp.sum(match * (r_of * 256.0 + c256), axis=1, keepdims=True)
    has = jnp.sum(match, axis=1, keepdims=True) > 0.5
    tcol = tF
    cv = jnp.where(has, cv, -3.0e38)
    ce = jnp.where(has, ce, 1.0e7 + tcol)

    gt = ((cv.T > cv) | ((cv.T == cv) & (ce.T < ce))).astype(f32)
    rank = jnp.sum(gt, axis=1, keepdims=True)
    jF = lax.broadcasted_iota(i32, (NPAD, NPAD), 0).astype(f32)
    oneh_o = (rank.T == jF).astype(f32)
    eidx = jnp.dot(oneh_o, ce, precision=HI,
                   preferred_element_type=f32)
    ei = jnp.clip(eidx, 0.0, float(L * NSC - 1)).astype(jnp.int32)
    idx_ref[0] = ei
    gidx_ref[0] = b * L + ei // NSC
    gidx2_ref[0] = (b * L + ei // NSC) // 8


def _dec_body(g_ref, rw_ref, idx_ref, bw1_ref, bb1_ref, bw2_ref, bb2_ref,
              bw3_ref, bb3_ref, ew_ref, eb_ref, eg_ref, ebb_ref,
              oe_ref, orf_ref, op_ref):
    f32 = jnp.float32
    g = g_ref[...]
    idx = idx_ref[...]
    s = idx % NSC

    t = jnp.maximum(jnp.dot(g, bw1_ref[...], preferred_element_type=f32) + bb1_ref[...], 0.0)
    t = jnp.maximum(jnp.dot(t, bw2_ref[...], preferred_element_type=f32) + bb2_ref[...], 0.0)
    tmp16 = jnp.dot(t, bw3_ref[...], preferred_element_type=f32) + bb3_ref[...]
    rw128 = rw_ref[...]
    m8 = (idx // NSC) % 8
    rw16 = jnp.zeros_like(tmp16)
    for gi in range(8):
        pick = (m8 == gi).astype(f32)
        rw16 = rw16 + pick * rw128[:, 16 * gi:16 * gi + 16]
    tmp4 = jnp.zeros_like(tmp16[:, 0:NSC])
    rw4 = jnp.zeros_like(tmp4)
    for sc in range(NSC):
        pick = (s == sc).astype(f32)
        tmp4 = tmp4 + pick * tmp16[:, NSC * sc:NSC * sc + NSC]
        rw4 = rw4 + pick * rw16[:, NSC * sc:NSC * sc + NSC]
    rwc = jnp.clip(rw4, 1e-5, 1.0 - 1e-5)
    x = tmp4 + jnp.log(rwc / (1.0 - rwc))
    oref = 1.0 / (1.0 + jnp.exp(-x))
    orf_ref[...] = oref

    ch = lax.broadcasted_iota(jnp.int32, (1, 128), 1)
    expo = (2 * (ch // 2)).astype(f32) / 128.0
    invd = jnp.exp(-expo * math.log(10000.0))
    even = (ch % 2) == 0

    def half(p):
        ang = (p * (2.0 * math.pi)) * invd
        return jnp.where(even, jnp.sin(ang), jnp.cos(ang))

    hx = half(oref[:, 0:1])
    hy = half(oref[:, 1:2])
    hw = half(oref[:, 2:3])
    hh = half(oref[:, 3:4])
    op_ref[...] = jnp.concatenate([hx + hw, hy + hh], axis=1)

    y = jnp.dot(g, ew_ref[...], preferred_element_type=f32) + eb_ref[...]
    mu = jnp.mean(y, axis=-1, keepdims=True)
    var = jnp.mean((y - mu) ** 2, axis=-1, keepdims=True)
    oe_ref[...] = (y - mu) / jnp.sqrt(var + 1e-5) * eg_ref[...] + ebb_ref[...]


def _gather_sc(out2, rw2, gidx, gidx2):
    bn = B * NPAD
    nw = 32
    per = bn // nw
    mesh = plsc.VectorSubcoreMesh(core_axis_name="c", subcore_axis_name="s")

    @functools.partial(
        pl.kernel, mesh=mesh,
        out_type=[jax.ShapeDtypeStruct((bn, D), jnp.float32),
                  jax.ShapeDtypeStruct((bn, 128), jnp.float32)],
        scratch_types=[pltpu.VMEM((per,), jnp.int32),
                       pltpu.VMEM((per,), jnp.int32),
                       pltpu.VMEM((per, D), jnp.float32),
                       pltpu.VMEM((per, 128), jnp.float32),
                       pltpu.SemaphoreType.DMA,
                       pltpu.SemaphoreType.DMA],
    )
    def k(out2_hbm, rw2_hbm, gidx_hbm, gidx2_hbm, o1_hbm, o2_hbm,
          idx_v, idx2_v, rows_v, rws_v, sem1, sem2):
        wid = lax.axis_index("s") * 2 + lax.axis_index("c")
        base = wid * per
        pltpu.sync_copy(gidx_hbm.at[pl.ds(base, per)], idx_v)
        pltpu.sync_copy(gidx2_hbm.at[pl.ds(base, per)], idx2_v)
        cp1 = pltpu.async_copy(out2_hbm.at[idx_v], rows_v, sem1)
        cp2 = pltpu.async_copy(rw2_hbm.at[idx2_v], rws_v, sem2)
        cp1.wait()
        cp2.wait()
        pltpu.sync_copy(rows_v, o1_hbm.at[pl.ds(base, per)])
        pltpu.sync_copy(rws_v, o2_hbm.at[pl.ds(base, per)])

    return k(out2, rw2, gidx, gidx2)


def kernel(src, pos, src_shape, src_mask, src_start_index, src_valid_ratios,
           ref_windows, W1, b1, W2, b2, ln_g, ln_b, bbox_w1, bbox_b1, bbox_w2,
           bbox_b2, bbox_w3, bbox_b3, cls_w, cls_b, enc_w, enc_b, enc_ln_g,
           enc_ln_b):
    f32 = jnp.float32
    bl = B * L
    src2 = src.reshape(bl, D)
    pos2 = pos.reshape(bl, D)
    msk2 = src_mask.astype(f32).reshape(bl, 1)
    row = lambda a: a.reshape(1, -1)
    nt = bl // TA

    out2, lg2 = pl.pallas_call(
        _enc_body,
        grid=(nt,),
        in_specs=[
            pl.BlockSpec((TA, D), lambda i: (i, 0)),
            pl.BlockSpec((TA, D), lambda i: (i, 0)),
            pl.BlockSpec((TA, 1), lambda i: (i, 0)),
            pl.BlockSpec((D, FFN), lambda i: (0, 0)),
            pl.BlockSpec((1, FFN), lambda i: (0, 0)),
            pl.BlockSpec((FFN, D), lambda i: (0, 0)),
            pl.BlockSpec((1, D), lambda i: (0, 0)),
            pl.BlockSpec((1, D), lambda i: (0, 0)),
            pl.BlockSpec((1, D), lambda i: (0, 0)),
            pl.BlockSpec((D, NSC), lambda i: (0, 0)),
            pl.BlockSpec((1, NSC), lambda i: (0, 0)),
        ],
        out_specs=[
            pl.BlockSpec((TA, D), lambda i: (i, 0)),
            pl.BlockSpec((TA, NSC), lambda i: (i, 0)),
        ],
        out_shape=[
            jax.ShapeDtypeStruct((bl, D), f32),
            jax.ShapeDtypeStruct((bl, NSC), f32),
        ],
    )(src2, pos2, msk2, W1, row(b1), W2, row(b2), row(ln_g), row(ln_b),
      cls_w, row(cls_b))

    output = out2.reshape(B, L, D)
    lgr = lg2.reshape(B, 256, 256)

    idxs = pl.pallas_call(
        _topk_body,
        grid=(B,),
        in_specs=[pl.BlockSpec((1, 256, 256), lambda i: (i, 0, 0))],
        out_specs=[
            pl.BlockSpec((1, NPAD, 1), lambda i: (i, 0, 0)),
            pl.BlockSpec((1, NPAD, 1), lambda i: (i, 0, 0)),
            pl.BlockSpec((1, NPAD, 1), lambda i: (i, 0, 0)),
        ],
        out_shape=[
            jax.ShapeDtypeStruct((B, NPAD, 1), jnp.int32),
            jax.ShapeDtypeStruct((B, NPAD, 1), jnp.int32),
            jax.ShapeDtypeStruct((B, NPAD, 1), jnp.int32),
        ],
    )(lgr)

    idx, gidx, gidx2 = idxs

    g2, rwsel = _gather_sc(out2, ref_windows.reshape(bl // 8, 128),
                           gidx.reshape(B * NPAD), gidx2.reshape(B * NPAD))

    bn = B * NPAD
    td = NPAD
    oe, orf, op = pl.pallas_call(
        _dec_body,
        grid=(bn // td,),
        in_specs=[
            pl.BlockSpec((td, D), lambda i: (i, 0)),
            pl.BlockSpec((td, 128), lambda i: (i, 0)),
            pl.BlockSpec((td, 1), lambda i: (i, 0)),
            pl.BlockSpec((D, D), lambda i: (0, 0)),
            pl.BlockSpec((1, D), lambda i: (0, 0)),
            pl.BlockSpec((D, D), lambda i: (0, 0)),
            pl.BlockSpec((1, D), lambda i: (0, 0)),
            pl.BlockSpec((D, 16), lambda i: (0, 0)),
            pl.BlockSpec((1, 16), lambda i: (0, 0)),
            pl.BlockSpec((D, DEC), lambda i: (0, 0)),
            pl.BlockSpec((1, DEC), lambda i: (0, 0)),
            pl.BlockSpec((1, DEC), lambda i: (0, 0)),
            pl.BlockSpec((1, DEC), lambda i: (0, 0)),
        ],
        out_specs=[
            pl.BlockSpec((td, DEC), lambda i: (i, 0)),
            pl.BlockSpec((td, NSC), lambda i: (i, 0)),
            pl.BlockSpec((td, DEC), lambda i: (i, 0)),
        ],
        out_shape=[
            jax.ShapeDtypeStruct((bn, DEC), f32),
            jax.ShapeDtypeStruct((bn, NSC), f32),
            jax.ShapeDtypeStruct((bn, DEC), f32),
        ],
    )(g2, rwsel, idx.reshape(bn, 1), bbox_w1, row(bbox_b1), bbox_w2,
      row(bbox_b2), bbox_w3, row(bbox_b3), enc_w, row(enc_b), row(enc_ln_g),
      row(enc_ln_b))

    out_embed = oe.reshape(B, NPAD, DEC)[:, :NQ]
    out_ref = orf.reshape(B, NPAD, NSC)[:, :NQ]
    out_pos = op.reshape(B, NPAD, DEC)[:, :NQ]
    return (output, out_embed, out_ref, out_pos)


# speedup vs baseline: 2.0448x; 1.0033x over previous
import functools
import math

import jax
import jax.numpy as jnp
from jax import lax
from jax.experimental import pallas as pl
from jax.experimental.pallas import tpu as pltpu
from jax.experimental.pallas import tpu_sc as plsc

B, L, D, FFN, NSC, NQ, DEC = 4, 16384, 256, 1024, 4, 900, 256
NPAD = 1024
TA = 4096
HI = lax.Precision.HIGHEST
NEG = -65504.0


def _enc_body(src_ref, pos_ref, msk_ref, w1_ref, b1_ref, w2_ref, b2_ref,
              g_ref, bb_ref, cw_ref, cb_ref, out_ref, lg_ref):
    x = src_ref[...]
    q = x + pos_ref[...]
    h = jnp.maximum(jnp.dot(q, w1_ref[...], preferred_element_type=jnp.float32) + b1_ref[...], 0.0)
    h = jnp.dot(h, w2_ref[...], preferred_element_type=jnp.float32) + b2_ref[...]
    y = x + h
    yt = y.T
    mu = (jnp.sum(yt, axis=0, keepdims=True) / float(D)).T
    var = (jnp.sum((yt - mu.T) ** 2, axis=0, keepdims=True) / float(D)).T
    yn = (y - mu) / jnp.sqrt(var + 1e-5) * g_ref[...] + bb_ref[...]
    m = msk_ref[...]
    yn = jnp.where(m > 0.5, 0.0, yn)
    out_ref[...] = yn
    lg = jnp.dot(yn, cw_ref[...],  preferred_element_type=jnp.float32) + cb_ref[...]
    lg_ref[...] = jnp.where(m > 0.5, NEG, lg)


def _topk_body(lg_ref, idx_ref, gidx_ref, gidx2_ref):
    b = pl.program_id(0)
    i32 = jnp.int32
    f32 = jnp.float32
    v = lg_ref[0]
    kraw = lax.bitcast_convert_type(v, i32)
    key = jnp.where(kraw < 0, kraw ^ i32(0x7FFFFFFF), kraw)
    MINI = i32(-2147483648)

    def bit_step(j, tu):
        cand = tu | (i32(1) << (31 - j))
        cnt = jnp.sum((key >= (cand ^ MINI)).astype(i32))
        return jnp.where(cnt >= NQ, cand, tu)

    tu = lax.fori_loop(0, 32, bit_step, i32(0))
    kth = tu ^ MINI
    n1 = jnp.sum((key > kth).astype(i32))
    tneed = NQ - n1

    rr = lax.broadcasted_iota(i32, (256, 256), 0)
    cc = lax.broadcasted_iota(i32, (256, 256), 1)
    e = rr * 256 + cc
    tie = key == kth

    def e_step(_, lohi):
        lo, hi = lohi
        mid = (lo + hi) // 2
        cnt = jnp.sum((tie & (e <= mid)).astype(i32))
        ok = cnt >= tneed
        return (jnp.where(ok, lo, mid), jnp.where(ok, mid, hi))

    _, ecut = lax.fori_loop(0, 20, e_step, (i32(-2), i32(L * NSC - 1)))
    sel = (key > kth) | (tie & (e <= ecut))
    selF = sel.astype(f32)

    ustrict = (lax.broadcasted_iota(i32, (256, 256), 0)
               < lax.broadcasted_iota(i32, (256, 256), 1)).astype(f32)
    prow = jnp.dot(selF, ustrict, precision=HI,
                   preferred_element_type=f32)
    rowsum = jnp.sum(selF, axis=1, keepdims=True)
    lstrict = (lax.broadcasted_iota(i32, (256, 256), 0)
               > lax.broadcasted_iota(i32, (256, 256), 1)).astype(f32)
    rowoff = jnp.dot(lstrict, rowsum, precision=HI,
                     preferred_element_type=f32)

    tF = lax.broadcasted_iota(i32, (NPAD, 1), 0).astype(f32)
    r_of = jnp.sum((rowoff.T <= tF).astype(f32), axis=1, keepdims=True) - 1.0
    c256 = lax.broadcasted_iota(i32, (NPAD, 256), 1).astype(f32)
    oneh_r = (c256 == r_of).astype(f32)
    rowoff_t = jnp.dot(oneh_r, rowoff, precision=HI,
                       preferred_element_type=f32)
    k_t = tF - rowoff_t
    prow_t = jnp.dot(oneh_r, prow, precision=HI, preferred_element_type=f32)
    sel_t = jnp.dot(oneh_r, selF, precision=HI, preferred_element_type=f32)
    val_t = jnp.dot(oneh_r, v, precision=HI, preferred_element_type=f32)
    match = ((prow_t == k_t) & (sel_t > 0.5)).astype(f32)
    cv = jnp.sum(match * val_t, axis=1, keepdims=True)
    ce = jnp.sum(match * (r_of * 256.0 + c256), axis=1, keepdims=True)
    has = jnp.sum(match, axis=1, keepdims=True) > 0.5
    tcol = tF
    cv = jnp.where(has, cv, -3.0e38)
    ce = jnp.where(has, ce, 1.0e7 + tcol)

    gt = ((cv.T > cv) | ((cv.T == cv) & (ce.T < ce))).astype(f32)
    rank = jnp.sum(gt, axis=1, keepdims=True)
    jF = lax.broadcasted_iota(i32, (NPAD, NPAD), 0).astype(f32)
    oneh_o = (rank.T == jF).astype(f32)
    eidx = jnp.dot(oneh_o, ce, precision=HI,
                   preferred_element_type=f32)
    ei = jnp.clip(eidx, 0.0, float(L * NSC - 1)).astype(jnp.int32)
    idx_ref[0] = ei
    gidx_ref[0] = b * L + ei // NSC
    gidx2_ref[0] = (b * L + ei // NSC) // 8


def _dec_body(g_ref, rw_ref, idx_ref, bw1_ref, bb1_ref, bw2_ref, bb2_ref,
              bw3_ref, bb3_ref, ew_ref, eb_ref, eg_ref, ebb_ref,
              oe_ref, orf_ref, op_ref):
    f32 = jnp.float32
    g = g_ref[...]
    idx = idx_ref[...]
    s = idx % NSC

    t = jnp.maximum(jnp.dot(g, bw1_ref[...], preferred_element_type=f32) + bb1_ref[...], 0.0)
    t = jnp.maximum(jnp.dot(t, bw2_ref[...], preferred_element_type=f32) + bb2_ref[...], 0.0)
    tmp16 = jnp.dot(t, bw3_ref[...], preferred_element_type=f32) + bb3_ref[...]
    rw128 = rw_ref[...]
    m8 = (idx // NSC) % 8
    rw16 = jnp.zeros_like(tmp16)
    for gi in range(8):
        pick = (m8 == gi).astype(f32)
        rw16 = rw16 + pick * rw128[:, 16 * gi:16 * gi + 16]
    tmp4 = jnp.zeros_like(tmp16[:, 0:NSC])
    rw4 = jnp.zeros_like(tmp4)
    for sc in range(NSC):
        pick = (s == sc).astype(f32)
        tmp4 = tmp4 + pick * tmp16[:, NSC * sc:NSC * sc + NSC]
        rw4 = rw4 + pick * rw16[:, NSC * sc:NSC * sc + NSC]
    rwc = jnp.clip(rw4, 1e-5, 1.0 - 1e-5)
    x = tmp4 + jnp.log(rwc / (1.0 - rwc))
    oref = 1.0 / (1.0 + jnp.exp(-x))
    orf_ref[...] = oref

    ch = lax.broadcasted_iota(jnp.int32, (1, 128), 1)
    expo = (2 * (ch // 2)).astype(f32) / 128.0
    invd = jnp.exp(-expo * math.log(10000.0))
    even = (ch % 2) == 0

    def half(p):
        ang = (p * (2.0 * math.pi)) * invd
        return jnp.where(even, jnp.sin(ang), jnp.cos(ang))

    hx = half(oref[:, 0:1])
    hy = half(oref[:, 1:2])
    hw = half(oref[:, 2:3])
    hh = half(oref[:, 3:4])
    op_ref[...] = jnp.concatenate([hx + hw, hy + hh], axis=1)

    y = jnp.dot(g, ew_ref[...], preferred_element_type=f32) + eb_ref[...]
    mu = jnp.mean(y, axis=-1, keepdims=True)
    var = jnp.mean((y - mu) ** 2, axis=-1, keepdims=True)
    oe_ref[...] = (y - mu) / jnp.sqrt(var + 1e-5) * eg_ref[...] + ebb_ref[...]


def _gather_sc(out2, rw2, gidx, gidx2):
    bn = B * NPAD
    nw = 32
    per = bn // nw
    mesh = plsc.VectorSubcoreMesh(core_axis_name="c", subcore_axis_name="s")

    @functools.partial(
        pl.kernel, mesh=mesh,
        out_type=[jax.ShapeDtypeStruct((bn, D), jnp.float32),
                  jax.ShapeDtypeStruct((bn, 128), jnp.float32)],
        scratch_types=[pltpu.VMEM((per,), jnp.int32),
                       pltpu.VMEM((per,), jnp.int32),
                       pltpu.VMEM((per, D), jnp.float32),
                       pltpu.VMEM((per, 128), jnp.float32),
                       pltpu.SemaphoreType.DMA,
                       pltpu.SemaphoreType.DMA],
    )
    def k(out2_hbm, rw2_hbm, gidx_hbm, gidx2_hbm, o1_hbm, o2_hbm,
          idx_v, idx2_v, rows_v, rws_v, sem1, sem2):
        wid = lax.axis_index("s") * 2 + lax.axis_index("c")
        base = wid * per
        pltpu.sync_copy(gidx_hbm.at[pl.ds(base, per)], idx_v)
        pltpu.sync_copy(gidx2_hbm.at[pl.ds(base, per)], idx2_v)
        cp1 = pltpu.async_copy(out2_hbm.at[idx_v], rows_v, sem1)
        cp2 = pltpu.async_copy(rw2_hbm.at[idx2_v], rws_v, sem2)
        cp1.wait()
        cp2.wait()
        pltpu.sync_copy(rows_v, o1_hbm.at[pl.ds(base, per)])
        pltpu.sync_copy(rws_v, o2_hbm.at[pl.ds(base, per)])

    return k(out2, rw2, gidx, gidx2)


def kernel(src, pos, src_shape, src_mask, src_start_index, src_valid_ratios,
           ref_windows, W1, b1, W2, b2, ln_g, ln_b, bbox_w1, bbox_b1, bbox_w2,
           bbox_b2, bbox_w3, bbox_b3, cls_w, cls_b, enc_w, enc_b, enc_ln_g,
           enc_ln_b):
    f32 = jnp.float32
    bl = B * L
    src2 = src.reshape(bl, D)
    pos2 = pos.reshape(bl, D)
    msk2 = src_mask.astype(f32).reshape(bl, 1)
    row = lambda a: a.reshape(1, -1)
    nt = bl // TA

    out2, lg2 = pl.pallas_call(
        _enc_body,
        grid=(nt,),
        in_specs=[
            pl.BlockSpec((TA, D), lambda i: (i, 0)),
            pl.BlockSpec((TA, D), lambda i: (i, 0)),
            pl.BlockSpec((TA, 1), lambda i: (i, 0)),
            pl.BlockSpec((D, FFN), lambda i: (0, 0)),
            pl.BlockSpec((1, FFN), lambda i: (0, 0)),
            pl.BlockSpec((FFN, D), lambda i: (0, 0)),
            pl.BlockSpec((1, D), lambda i: (0, 0)),
            pl.BlockSpec((1, D), lambda i: (0, 0)),
            pl.BlockSpec((1, D), lambda i: (0, 0)),
            pl.BlockSpec((D, NSC), lambda i: (0, 0)),
            pl.BlockSpec((1, NSC), lambda i: (0, 0)),
        ],
        out_specs=[
            pl.BlockSpec((TA, D), lambda i: (i, 0)),
            pl.BlockSpec((TA, NSC), lambda i: (i, 0)),
        ],
        out_shape=[
            jax.ShapeDtypeStruct((bl, D), f32),
            jax.ShapeDtypeStruct((bl, NSC), f32),
        ],
    )(src2, pos2, msk2, W1, row(b1), W2, row(b2), row(ln_g), row(ln_b),
      cls_w, row(cls_b))

    output = out2.reshape(B, L, D)
    lgr = lg2.reshape(B, 256, 256)

    idxs = pl.pallas_call(
        _topk_body,
        grid=(B,),
        in_specs=[pl.BlockSpec((1, 256, 256), lambda i: (i, 0, 0))],
        out_specs=[
            pl.BlockSpec((1, NPAD, 1), lambda i: (i, 0, 0)),
            pl.BlockSpec((1, NPAD, 1), lambda i: (i, 0, 0)),
            pl.BlockSpec((1, NPAD, 1), lambda i: (i, 0, 0)),
        ],
        out_shape=[
            jax.ShapeDtypeStruct((B, NPAD, 1), jnp.int32),
            jax.ShapeDtypeStruct((B, NPAD, 1), jnp.int32),
            jax.ShapeDtypeStruct((B, NPAD, 1), jnp.int32),
        ],
    )(lgr)

    idx, gidx, gidx2 = idxs

    g2, rwsel = _gather_sc(out2, ref_windows.reshape(bl // 8, 128),
                           gidx.reshape(B * NPAD), gidx2.reshape(B * NPAD))

    bn = B * NPAD
    td = NPAD
    oe, orf, op = pl.pallas_call(
        _dec_body,
        grid=(bn // td,),
        in_specs=[
            pl.BlockSpec((td, D), lambda i: (i, 0)),
            pl.BlockSpec((td, 128), lambda i: (i, 0)),
            pl.BlockSpec((td, 1), lambda i: (i, 0)),
            pl.BlockSpec((D, D), lambda i: (0, 0)),
            pl.BlockSpec((1, D), lambda i: (0, 0)),
            pl.BlockSpec((D, D), lambda i: (0, 0)),
            pl.BlockSpec((1, D), lambda i: (0, 0)),
            pl.BlockSpec((D, 16), lambda i: (0, 0)),
            pl.BlockSpec((1, 16), lambda i: (0, 0)),
            pl.BlockSpec((D, DEC), lambda i: (0, 0)),
            pl.BlockSpec((1, DEC), lambda i: (0, 0)),
            pl.BlockSpec((1, DEC), lambda i: (0, 0)),
            pl.BlockSpec((1, DEC), lambda i: (0, 0)),
        ],
        out_specs=[
            pl.BlockSpec((td, DEC), lambda i: (i, 0)),
            pl.BlockSpec((td, NSC), lambda i: (i, 0)),
            pl.BlockSpec((td, DEC), lambda i: (i, 0)),
        ],
        out_shape=[
            jax.ShapeDtypeStruct((bn, DEC), f32),
            jax.ShapeDtypeStruct((bn, NSC), f32),
            jax.ShapeDtypeStruct((bn, DEC), f32),
        ],
    )(g2, rwsel, idx.reshape(bn, 1), bbox_w1, row(bbox_b1), bbox_w2,
      row(bbox_b2), bbox_w3, row(bbox_b3), enc_w, row(enc_b), row(enc_ln_g),
      row(enc_ln_b))

    out_embed = oe.reshape(B, NPAD, DEC)[:, :NQ]
    out_ref = orf.reshape(B, NPAD, NSC)[:, :NQ]
    out_pos = op.reshape(B, NPAD, DEC)[:, :NQ]
    return (output, out_embed, out_ref, out_pos)
